# baseline (speedup 1.0000x reference)
import functools

import numpy as np
import jax
import jax.numpy as jnp
from jax import lax
from jax.experimental import pallas as pl
from jax.experimental.pallas import tpu as pltpu
from jax.experimental.pallas import tpu_sc as plsc

B, L, V, E, O = 4096, 50, 100000, 128, 10000

NCH = 2
B_CH = B // NCH

NC = 2
NS = 16
NW = NC * NS
CH_ROWS = B_CH * L
ROWS_PER_W = CH_ROWS // NW
CHUNK = 128
K_INFLIGHT = 5
OUTER = ROWS_PER_W // (CHUNK * K_INFLIGHT)

_sc_mesh = plsc.VectorSubcoreMesh(core_axis_name="c", subcore_axis_name="s")


@functools.partial(
    pl.kernel,
    mesh=_sc_mesh,
    out_type=jax.ShapeDtypeStruct((CH_ROWS, E), jnp.float32),
    scratch_types=[
        pltpu.VMEM((ROWS_PER_W,), jnp.int32),
        pltpu.VMEM((CHUNK * K_INFLIGHT, E), jnp.float32),
        pltpu.SemaphoreType.DMA,
    ],
)
def _sc_gather(idx_hbm, table_hbm, out_hbm, idx_v, rows_v, sem):
    wid = lax.axis_index("s") * NC + lax.axis_index("c")
    base = wid * ROWS_PER_W
    pltpu.sync_copy(idx_hbm.at[pl.ds(base, ROWS_PER_W)], idx_v)
    for outer in range(OUTER):
        o0 = outer * CHUNK * K_INFLIGHT
        copies = []
        for j in range(K_INFLIGHT):
            copies.append(
                pltpu.async_copy(
                    table_hbm.at[idx_v.at[pl.ds(o0 + j * CHUNK, CHUNK)]],
                    rows_v.at[pl.ds(j * CHUNK, CHUNK)],
                    sem,
                )
            )
        for cp in copies:
            cp.wait()
        pltpu.sync_copy(rows_v, out_hbm.at[pl.ds(base + o0, CHUNK * K_INFLIGHT)])


BBLK = 256
NTC = 8
B_TC = B // NTC
STEPS_CH = B_TC // BBLK
TC_PER_SC = NTC // NCH
GAMMA = np.float32(np.float32(0.9) * np.float32(L - 1) - np.float32(44.0))


def _tc_impl(g_ref, wt_ref, eb_ref, b_ref, out_ref):
    neg_inf = jnp.float32(-jnp.inf)
    top = [jnp.full((BBLK, E), neg_inf, jnp.float32) for _ in range(6)]
    for l in range(L):
        x = g_ref[l]
        for k in range(6):
            hi = jnp.maximum(top[k], x)
            x = jnp.minimum(top[k], x)
            top[k] = hi
    qs = top[5] + GAMMA * (top[4] - top[5])
    acc = jnp.zeros((BBLK, E), jnp.float32)
    for l in range(L):
        x = g_ref[l]
        acc = acc + jnp.where(x >= qs, x, 0.0)
    s = acc + eb_ref[...]
    out_ref[...] = (
        jnp.dot(s, wt_ref[...], preferred_element_type=jnp.float32)
        + b_ref[...]
    )


def _tc_body_first(g_ref, wt_ref, eb_ref, b_ref, out_ref):
    _tc_impl(g_ref, wt_ref, eb_ref, b_ref, out_ref)


def _tc_body_rest(g_ref, wt_ref, eb_ref, b_ref, prev_ref, out_ref):
    del prev_ref
    _tc_impl(g_ref, wt_ref, eb_ref, b_ref, out_ref)


def _make_tc_call(tcc):
    first = tcc == 0
    sub = tcc % TC_PER_SC
    base_specs = [
        pl.BlockSpec((L, BBLK, E),
                     lambda i, s=sub: (0, s * STEPS_CH + i, 0)),
        pl.BlockSpec((E, O), lambda i: (0, 0)),
        pl.BlockSpec((1, E), lambda i: (0, 0)),
        pl.BlockSpec((1, O), lambda i: (0, 0)),
    ]
    return pl.pallas_call(
        _tc_body_first if first else _tc_body_rest,
        grid=(STEPS_CH,),
        in_specs=base_specs if first else (
            base_specs + [pl.BlockSpec(memory_space=pltpu.MemorySpace.HBM)]
        ),
        out_specs=pl.BlockSpec(
            (BBLK, O), lambda i, c=tcc: (c * STEPS_CH + i, 0)
        ),
        out_shape=jax.ShapeDtypeStruct((B, O), jnp.float32),
        input_output_aliases={} if first else {4: 0},
    )


_tc_calls = [_make_tc_call(c) for c in range(NTC)]


def kernel(tokens, emb_table, emb_bias, W, b):
    idx_t = tokens.astype(jnp.int32).T
    wt = W.T
    eb = emb_bias.reshape(1, E)
    bb = b.reshape(1, O)
    gathered = []
    for c in range(NCH):
        idx_c = idx_t[:, c * B_CH:(c + 1) * B_CH].reshape(-1)
        gathered.append(_sc_gather(idx_c, emb_table).reshape(L, B_CH, E))
    out = _tc_calls[0](gathered[0], wt, eb, bb)
    for c in range(1, NTC):
        out = _tc_calls[c](gathered[c // TC_PER_SC], wt, eb, bb, out)
    return out


# speedup vs baseline: 1.1005x; 1.1005x over previous
import functools

import numpy as np
import jax
import jax.numpy as jnp
from jax import lax
from jax.experimental import pallas as pl
from jax.experimental.pallas import tpu as pltpu
from jax.experimental.pallas import tpu_sc as plsc

B, L, V, E, O = 4096, 50, 100000, 128, 10000

NC = 2
NS = 16
NW = NC * NS
N_ROWS = B * L
ROWS_PER_W = N_ROWS // NW
CHUNK = 128
K_INFLIGHT = 2
BUF_ROWS = CHUNK * K_INFLIGHT
OUTER = ROWS_PER_W // BUF_ROWS

_sc_mesh = plsc.VectorSubcoreMesh(core_axis_name="c", subcore_axis_name="s")


@functools.partial(
    pl.kernel,
    mesh=_sc_mesh,
    out_type=jax.ShapeDtypeStruct((N_ROWS, E), jnp.float32),
    scratch_types=[
        pltpu.VMEM((ROWS_PER_W,), jnp.int32),
        pltpu.VMEM((2, BUF_ROWS, E), jnp.float32),
        pltpu.SemaphoreType.DMA,
        pltpu.SemaphoreType.DMA,
    ],
)
def _sc_gather(idx_hbm, table_hbm, out_hbm, idx_v, rows_v, gsem, wsem):
    wid = lax.axis_index("s") * NC + lax.axis_index("c")
    base = wid * ROWS_PER_W
    pltpu.sync_copy(idx_hbm.at[pl.ds(base, ROWS_PER_W)], idx_v)
    for outer in range(OUTER):
        p = outer % 2
        o0 = outer * BUF_ROWS
        if outer >= 2:
            prev0 = (outer - 2) * BUF_ROWS
            pltpu.make_async_copy(
                rows_v.at[p], out_hbm.at[pl.ds(base + prev0, BUF_ROWS)], wsem
            ).wait()
        gathers = []
        for j in range(K_INFLIGHT):
            gathers.append(
                pltpu.async_copy(
                    table_hbm.at[idx_v.at[pl.ds(o0 + j * CHUNK, CHUNK)]],
                    rows_v.at[p, pl.ds(j * CHUNK, CHUNK)],
                    gsem,
                )
            )
        for g in gathers:
            g.wait()
        pltpu.async_copy(
            rows_v.at[p], out_hbm.at[pl.ds(base + o0, BUF_ROWS)], wsem
        )
    for tail in (OUTER - 2, OUTER - 1):
        pltpu.make_async_copy(
            rows_v.at[tail % 2],
            out_hbm.at[pl.ds(base + tail * BUF_ROWS, BUF_ROWS)],
            wsem,
        ).wait()


BBLK = 256
GAMMA = np.float32(np.float32(0.9) * np.float32(L - 1) - np.float32(44.0))


def _tc_body(g_ref, wt_ref, eb_ref, b_ref, out_ref):
    neg_inf = jnp.float32(-jnp.inf)
    top = [jnp.full((BBLK, E), neg_inf, jnp.float32) for _ in range(6)]
    for l in range(L):
        x = g_ref[l]
        for k in range(6):
            hi = jnp.maximum(top[k], x)
            x = jnp.minimum(top[k], x)
            top[k] = hi
    qs = top[5] + GAMMA * (top[4] - top[5])
    acc = jnp.zeros((BBLK, E), jnp.float32)
    for l in range(L):
        x = g_ref[l]
        acc = acc + jnp.where(x >= qs, x, 0.0)
    s = acc + eb_ref[...]
    out_ref[...] = (
        jnp.dot(s, wt_ref[...], preferred_element_type=jnp.float32)
        + b_ref[...]
    )


_tc_call = pl.pallas_call(
    _tc_body,
    grid=(B // BBLK,),
    in_specs=[
        pl.BlockSpec((L, BBLK, E), lambda i: (0, i, 0)),
        pl.BlockSpec((E, O), lambda i: (0, 0)),
        pl.BlockSpec((1, E), lambda i: (0, 0)),
        pl.BlockSpec((1, O), lambda i: (0, 0)),
    ],
    out_specs=pl.BlockSpec((BBLK, O), lambda i: (i, 0)),
    out_shape=jax.ShapeDtypeStruct((B, O), jnp.float32),
)


def kernel(tokens, emb_table, emb_bias, W, b):
    idx = tokens.astype(jnp.int32).T.reshape(-1)
    gathered = _sc_gather(idx, emb_table)
    g3 = gathered.reshape(L, B, E)
    return _tc_call(g3, W.T, emb_bias.reshape(1, E), b.reshape(1, O))


# speedup vs baseline: 1.1101x; 1.0087x over previous
import functools

import numpy as np
import jax
import jax.numpy as jnp
from jax import lax
from jax.experimental import pallas as pl
from jax.experimental.pallas import tpu as pltpu
from jax.experimental.pallas import tpu_sc as plsc

B, L, V, E, O = 4096, 50, 100000, 128, 10000

NC = 2
NS = 16
NW = NC * NS
N_ROWS = B * L
ROWS_PER_W = N_ROWS // NW
CHUNK = 128
K_INFLIGHT = 2
BUF_ROWS = CHUNK * K_INFLIGHT
OUTER = ROWS_PER_W // BUF_ROWS

_sc_mesh = plsc.VectorSubcoreMesh(core_axis_name="c", subcore_axis_name="s")


@functools.partial(
    pl.kernel,
    mesh=_sc_mesh,
    out_type=jax.ShapeDtypeStruct((N_ROWS, E), jnp.float32),
    scratch_types=[
        pltpu.VMEM((ROWS_PER_W,), jnp.int32),
        pltpu.VMEM((2, BUF_ROWS, E), jnp.float32),
        pltpu.SemaphoreType.DMA,
        pltpu.SemaphoreType.DMA,
    ],
)
def _sc_gather(idx_hbm, table_hbm, out_hbm, idx_v, rows_v, gsem, wsem):
    wid = lax.axis_index("s") * NC + lax.axis_index("c")
    base = wid * ROWS_PER_W
    pltpu.sync_copy(idx_hbm.at[pl.ds(base, ROWS_PER_W)], idx_v)
    for outer in range(OUTER):
        p = outer % 2
        o0 = outer * BUF_ROWS
        if outer >= 2:
            prev0 = (outer - 2) * BUF_ROWS
            pltpu.make_async_copy(
                rows_v.at[p], out_hbm.at[pl.ds(base + prev0, BUF_ROWS)], wsem
            ).wait()
        gathers = []
        for j in range(K_INFLIGHT):
            gathers.append(
                pltpu.async_copy(
                    table_hbm.at[idx_v.at[pl.ds(o0 + j * CHUNK, CHUNK)]],
                    rows_v.at[p, pl.ds(j * CHUNK, CHUNK)],
                    gsem,
                )
            )
        for g in gathers:
            g.wait()
        pltpu.async_copy(
            rows_v.at[p], out_hbm.at[pl.ds(base + o0, BUF_ROWS)], wsem
        )
    for tail in (OUTER - 2, OUTER - 1):
        pltpu.make_async_copy(
            rows_v.at[tail % 2],
            out_hbm.at[pl.ds(base + tail * BUF_ROWS, BUF_ROWS)],
            wsem,
        ).wait()


BBLK = 256
GAMMA = np.float32(np.float32(0.9) * np.float32(L - 1) - np.float32(44.0))


def _tc_body(g_ref, wt_ref, eb_ref, b_ref, out_ref, ob, wsem):
    i = pl.program_id(0)
    nsteps = pl.num_programs(0)
    par = i % 2

    @pl.when(i >= 2)
    def _drain():
        pltpu.make_async_copy(
            ob.at[par], out_ref.at[pl.ds((i - 2) * BBLK, BBLK)], wsem
        ).wait()

    neg_inf = jnp.float32(-jnp.inf)
    top = [jnp.full((BBLK, E), neg_inf, jnp.float32) for _ in range(6)]
    for l in range(L):
        x = g_ref[l]
        for k in range(6):
            hi = jnp.maximum(top[k], x)
            x = jnp.minimum(top[k], x)
            top[k] = hi
    qs = top[5] + GAMMA * (top[4] - top[5])
    acc = jnp.zeros((BBLK, E), jnp.float32)
    for l in range(L):
        x = g_ref[l]
        acc = acc + jnp.where(x >= qs, x, 0.0)
    s = acc + eb_ref[...]
    ob[par] = (
        jnp.dot(s, wt_ref[...], preferred_element_type=jnp.float32)
        + b_ref[...]
    )
    cp = pltpu.make_async_copy(
        ob.at[par], out_ref.at[pl.ds(i * BBLK, BBLK)], wsem
    )
    cp.start()

    @pl.when(i == nsteps - 1)
    def _tail():
        pltpu.make_async_copy(
            ob.at[1 - par], out_ref.at[pl.ds((i - 1) * BBLK, BBLK)], wsem
        ).wait()
        pltpu.make_async_copy(
            ob.at[par], out_ref.at[pl.ds(i * BBLK, BBLK)], wsem
        ).wait()


_tc_call = pl.pallas_call(
    _tc_body,
    grid=(B // BBLK,),
    in_specs=[
        pl.BlockSpec((L, BBLK, E), lambda i: (0, i, 0)),
        pl.BlockSpec((E, O), lambda i: (0, 0)),
        pl.BlockSpec((1, E), lambda i: (0, 0)),
        pl.BlockSpec((1, O), lambda i: (0, 0)),
    ],
    out_specs=pl.BlockSpec(memory_space=pltpu.MemorySpace.HBM),
    out_shape=jax.ShapeDtypeStruct((B, O), jnp.float32),
    scratch_shapes=[
        pltpu.VMEM((2, BBLK, O), jnp.float32),
        pltpu.SemaphoreType.DMA,
    ],
)


def kernel(tokens, emb_table, emb_bias, W, b):
    idx = tokens.astype(jnp.int32).T.reshape(-1)
    gathered = _sc_gather(idx, emb_table)
    g3 = gathered.reshape(L, B, E)
    return _tc_call(g3, W.T, emb_bias.reshape(1, E), b.reshape(1, O))
